# Initial kernel scaffold; baseline (speedup 1.0000x reference)
import jax
import jax.numpy as jnp
from jax.experimental import pallas as pl


def kernel(x, edge_attr, Wn, bn, We, be, W11, b11, W12, b12, W21, b21, W22, b22):
    raise NotImplementedError("write your pallas kernel here")


# speedup vs baseline: 21.9493x; 21.9493x over previous
import functools

import jax
import jax.numpy as jnp
from jax.experimental import pallas as pl

_N = 1024
_BI = 128
_BJ = 128
_PREC = jax.lax.Precision.HIGHEST


def _conv_kernel(feat_ref, e_ref, emb_w_ref, emb_b_ref, we_ref, be_ref,
                 w1at_ref, w1bt_ref, b1_ref, w2t_ref, b2_ref, out_ref,
                 *, embed_input: bool, bi: int, bj: int):
    i = pl.program_id(1)

    v = feat_ref[...]
    if embed_input:
        v = jnp.maximum(
            jnp.dot(v, emb_w_ref[...], precision=_PREC) + emb_b_ref[...], 0.0)
    a = jnp.dot(v, w1at_ref[...], precision=_PREC) + b1_ref[...]

    e = e_ref[...]
    we = we_ref[...]
    be = be_ref[...]
    ef = jnp.maximum(e[:, :, None] * we[0][None, None, :]
                     + be[0][None, None, :], 0.0)
    t = jnp.dot(ef.reshape(bi * bj, 16), w1bt_ref[...],
                precision=_PREC)
    o1 = jnp.maximum(t.reshape(bi, bj, 64) + a[:, None, :], 0.0)
    o2 = jnp.maximum(
        jnp.dot(o1.reshape(bi * bj, 64), w2t_ref[...], precision=_PREC)
        + b2_ref[...], 0.0)
    contrib = o2.reshape(bi, bj, 64).sum(axis=0)

    @pl.when(i == 0)
    def _():
        out_ref[...] = contrib

    @pl.when(i != 0)
    def _():
        out_ref[...] += contrib


def _conv_layer(feat, e2d, emb_w, emb_b, we_row, be_row,
                w1at, w1bt, b1_row, w2t, b2_row, embed_input):
    n = e2d.shape[0]
    f = feat.shape[1]
    grid = (n // _BJ, n // _BI)
    kern = functools.partial(_conv_kernel, embed_input=embed_input,
                             bi=_BI, bj=_BJ)
    return pl.pallas_call(
        kern,
        grid=grid,
        in_specs=[
            pl.BlockSpec((_BI, f), lambda j, i: (i, 0)),
            pl.BlockSpec((_BI, _BJ), lambda j, i: (i, j)),
            pl.BlockSpec(emb_w.shape, lambda j, i: (0, 0)),
            pl.BlockSpec(emb_b.shape, lambda j, i: (0, 0)),
            pl.BlockSpec(we_row.shape, lambda j, i: (0, 0)),
            pl.BlockSpec(be_row.shape, lambda j, i: (0, 0)),
            pl.BlockSpec(w1at.shape, lambda j, i: (0, 0)),
            pl.BlockSpec(w1bt.shape, lambda j, i: (0, 0)),
            pl.BlockSpec(b1_row.shape, lambda j, i: (0, 0)),
            pl.BlockSpec(w2t.shape, lambda j, i: (0, 0)),
            pl.BlockSpec(b2_row.shape, lambda j, i: (0, 0)),
        ],
        out_specs=pl.BlockSpec((_BJ, 64), lambda j, i: (j, 0)),
        out_shape=jax.ShapeDtypeStruct((n, 64), jnp.float32),
    )(feat, e2d, emb_w, emb_b, we_row, be_row,
      w1at, w1bt, b1_row, w2t, b2_row)


def kernel(x, edge_attr, Wn, bn, We, be, W11, b11, W12, b12, W21, b21,
           W22, b22):
    n = x.shape[0]
    x2 = x.reshape(n, 1)
    e2d = edge_attr.reshape(n, n)
    wn_t = Wn.T
    bn_r = bn.reshape(1, -1)
    we_r = We.reshape(1, -1)
    be_r = be.reshape(1, -1)
    w11at = W11[:, :64].T
    w11bt = W11[:, 64:].T
    b11_r = b11.reshape(1, -1)
    w12t = W12.T
    b12_r = b12.reshape(1, -1)
    w21at = W21[:, :64].T
    w21bt = W21[:, 64:].T
    b21_r = b21.reshape(1, -1)
    w22t = W22.T
    b22_r = b22.reshape(1, -1)

    h = _conv_layer(x2, e2d, wn_t, bn_r, we_r, be_r,
                    w11at, w11bt, b11_r, w12t, b12_r, embed_input=True)
    out = _conv_layer(h, e2d, wn_t, bn_r, we_r, be_r,
                      w21at, w21bt, b21_r, w22t, b22_r, embed_input=False)
    return out


# speedup vs baseline: 187.4215x; 8.5388x over previous
import functools

import jax
import jax.numpy as jnp
from jax.experimental import pallas as pl

_N = 1024
_BI = 128
_BJ = 128
_PREC = jax.lax.Precision.DEFAULT


def _conv_kernel(feat_ref, e_ref, emb_w_ref, emb_b_ref, we_ref, be_ref,
                 w1at_ref, w1bt_ref, b1_ref, w2t_ref, b2_ref, out_ref,
                 *, embed_input: bool, bi: int, bj: int):
    i = pl.program_id(1)

    v = feat_ref[...]
    if embed_input:
        v = jnp.maximum(
            jnp.dot(v, emb_w_ref[...], precision=_PREC) + emb_b_ref[...], 0.0)
    a = jnp.dot(v, w1at_ref[...], precision=_PREC) + b1_ref[...]

    e = e_ref[...]
    we = we_ref[...]
    be = be_ref[...]
    ef = jnp.maximum(e[:, :, None] * we[0][None, None, :]
                     + be[0][None, None, :], 0.0)
    t = jnp.dot(ef.reshape(bi * bj, 16), w1bt_ref[...],
                precision=_PREC)
    o1 = jnp.maximum(t.reshape(bi, bj, 64) + a[:, None, :], 0.0)
    o2 = jnp.maximum(
        jnp.dot(o1.reshape(bi * bj, 64), w2t_ref[...], precision=_PREC)
        + b2_ref[...], 0.0)
    contrib = o2.reshape(bi, bj, 64).sum(axis=0)

    @pl.when(i == 0)
    def _():
        out_ref[...] = contrib

    @pl.when(i != 0)
    def _():
        out_ref[...] += contrib


def _conv_layer(feat, e2d, emb_w, emb_b, we_row, be_row,
                w1at, w1bt, b1_row, w2t, b2_row, embed_input):
    n = e2d.shape[0]
    f = feat.shape[1]
    grid = (n // _BJ, n // _BI)
    kern = functools.partial(_conv_kernel, embed_input=embed_input,
                             bi=_BI, bj=_BJ)
    return pl.pallas_call(
        kern,
        grid=grid,
        in_specs=[
            pl.BlockSpec((_BI, f), lambda j, i: (i, 0)),
            pl.BlockSpec((_BI, _BJ), lambda j, i: (i, j)),
            pl.BlockSpec(emb_w.shape, lambda j, i: (0, 0)),
            pl.BlockSpec(emb_b.shape, lambda j, i: (0, 0)),
            pl.BlockSpec(we_row.shape, lambda j, i: (0, 0)),
            pl.BlockSpec(be_row.shape, lambda j, i: (0, 0)),
            pl.BlockSpec(w1at.shape, lambda j, i: (0, 0)),
            pl.BlockSpec(w1bt.shape, lambda j, i: (0, 0)),
            pl.BlockSpec(b1_row.shape, lambda j, i: (0, 0)),
            pl.BlockSpec(w2t.shape, lambda j, i: (0, 0)),
            pl.BlockSpec(b2_row.shape, lambda j, i: (0, 0)),
        ],
        out_specs=pl.BlockSpec((_BJ, 64), lambda j, i: (j, 0)),
        out_shape=jax.ShapeDtypeStruct((n, 64), jnp.float32),
    )(feat, e2d, emb_w, emb_b, we_row, be_row,
      w1at, w1bt, b1_row, w2t, b2_row)


def kernel(x, edge_attr, Wn, bn, We, be, W11, b11, W12, b12, W21, b21,
           W22, b22):
    n = x.shape[0]
    x2 = x.reshape(n, 1)
    e2d = edge_attr.reshape(n, n)
    wn_t = Wn.T
    bn_r = bn.reshape(1, -1)
    we_r = We.reshape(1, -1)
    be_r = be.reshape(1, -1)
    w11at = W11[:, :64].T
    w11bt = W11[:, 64:].T
    b11_r = b11.reshape(1, -1)
    w12t = W12.T
    b12_r = b12.reshape(1, -1)
    w21at = W21[:, :64].T
    w21bt = W21[:, 64:].T
    b21_r = b21.reshape(1, -1)
    w22t = W22.T
    b22_r = b22.reshape(1, -1)

    h = _conv_layer(x2, e2d, wn_t, bn_r, we_r, be_r,
                    w11at, w11bt, b11_r, w12t, b12_r, embed_input=True)
    out = _conv_layer(h, e2d, wn_t, bn_r, we_r, be_r,
                      w21at, w21bt, b21_r, w22t, b22_r, embed_input=False)
    return out
